# Initial kernel scaffold; baseline (speedup 1.0000x reference)
import jax
import jax.numpy as jnp
from jax.experimental import pallas as pl


def kernel(x, edge_index, batch, W0, b0, Wr0, br0, g0, be0, W1, b1, Wr1, br1, g1, be1, W2, b2, Wr2, br2, g2, be2):
    raise NotImplementedError("write your pallas kernel here")


# speedup vs baseline: 10.2213x; 10.2213x over previous
import functools

import jax
import jax.numpy as jnp
from jax import lax
from jax.experimental import pallas as pl
from jax.experimental.pallas import tpu as pltpu
from jax.experimental.pallas import tpu_sc as plsc

N = 10000
E = 320000
D = 128
G = 64
EPS = 1e-5

NC, NS = 2, 16
NW = NC * NS
EPW = E // NW
CH = 80
NCHUNK = EPW // CH
ZCH = 80
NZ = N // ZCH
KZ = -(-NZ // NS)

RB = 400
NB = N // RB

_mesh = plsc.VectorSubcoreMesh(core_axis_name="c", subcore_axis_name="s")


def _deg_body(dst_hbm, out_hbm, dst_v, ones_v, st_v, acc_sh):
    cid = lax.axis_index("c")
    sid = lax.axis_index("s")
    ebase = (cid * NS + sid) * EPW

    one16 = jnp.full((16,), 1.0, dtype=jnp.float32)
    zero16 = jnp.zeros((16,), dtype=jnp.float32)

    def fill_ones(i, _):
        ones_v[i, :] = one16
        return 0
    lax.fori_loop(0, CH, fill_ones, 0)

    def fill_zero(i, _):
        st_v[i, :] = zero16
        return 0
    lax.fori_loop(0, ZCH, fill_zero, 0)

    for k in range(KZ):
        c = sid + k * NS

        @pl.when(c < NZ)
        def _():
            pltpu.sync_copy(st_v, acc_sh.at[pl.ds(c * ZCH, ZCH), :])
    plsc.subcore_barrier()

    def step(g, _):
        pltpu.sync_copy(dst_hbm.at[pl.ds(ebase + g * CH, CH)], dst_v)
        pltpu.sync_copy(ones_v, acc_sh.at[dst_v], add=True)
        return 0
    lax.fori_loop(0, NCHUNK, step, 0)
    plsc.subcore_barrier()

    for k in range(KZ):
        c = sid + k * NS

        @pl.when(c < NZ)
        def _():
            pltpu.sync_copy(acc_sh.at[pl.ds(c * ZCH, ZCH), :], st_v)
            pltpu.sync_copy(st_v, out_hbm.at[cid, pl.ds(c * ZCH, ZCH), :])


_deg = functools.partial(
    pl.kernel,
    out_type=jax.ShapeDtypeStruct((NC, N, 16), jnp.float32),
    mesh=_mesh,
    scratch_types=[
        pltpu.VMEM((CH,), jnp.int32),
        pltpu.VMEM((CH, 16), jnp.float32),
        pltpu.VMEM((ZCH, 16), jnp.float32),
        pltpu.VMEM_SHARED((N, 16), jnp.float32),
    ],
)(_deg_body)


def _agg_body(y_hbm, src_hbm, dst_hbm, out_hbm, src_v, dst_v, rows_v, st_v,
              acc_sh, sem):
    cid = lax.axis_index("c")
    sid = lax.axis_index("s")
    ebase = (cid * NS + sid) * EPW

    zero16 = jnp.zeros((16,), dtype=jnp.float32)

    def fill_zero(i, _):
        for j in range(D // 16):
            st_v[i, pl.ds(j * 16, 16)] = zero16
        return 0
    lax.fori_loop(0, ZCH, fill_zero, 0)

    for k in range(KZ):
        c = sid + k * NS

        @pl.when(c < NZ)
        def _():
            pltpu.sync_copy(st_v, acc_sh.at[pl.ds(c * ZCH, ZCH), :])
    plsc.subcore_barrier()

    def step(g, _):
        pltpu.sync_copy(src_hbm.at[pl.ds(ebase + g * CH, CH)], src_v)
        pltpu.sync_copy(dst_hbm.at[pl.ds(ebase + g * CH, CH)], dst_v)
        pltpu.async_copy(y_hbm.at[src_v], rows_v, sem).wait()
        pltpu.sync_copy(rows_v, acc_sh.at[dst_v], add=True)
        return 0
    lax.fori_loop(0, NCHUNK, step, 0)
    plsc.subcore_barrier()

    for k in range(KZ):
        c = sid + k * NS

        @pl.when(c < NZ)
        def _():
            pltpu.sync_copy(acc_sh.at[pl.ds(c * ZCH, ZCH), :], st_v)
            pltpu.sync_copy(st_v, out_hbm.at[cid, pl.ds(c * ZCH, ZCH), :])


_agg = functools.partial(
    pl.kernel,
    out_type=jax.ShapeDtypeStruct((NC, N, D), jnp.float32),
    mesh=_mesh,
    scratch_types=[
        pltpu.VMEM((CH,), jnp.int32),
        pltpu.VMEM((CH,), jnp.int32),
        pltpu.VMEM((CH, D), jnp.float32),
        pltpu.VMEM((ZCH, D), jnp.float32),
        pltpu.VMEM_SHARED((N, D), jnp.float32),
        pltpu.SemaphoreType.DMA,
    ],
)(_agg_body)


def _dis_block(degp):
    deg = degp[0, :, 0] + degp[1, :, 0] + 1.0
    return lax.rsqrt(deg)


def _pre_body(h_ref, wc_ref, br_ref, degp_ref, y_ref, r_ref):
    z = jnp.dot(h_ref[...], wc_ref[...], preferred_element_type=jnp.float32)
    dis = _dis_block(degp_ref[...])
    y_ref[...] = z[:, :D] * dis[:, None]
    r_ref[...] = jnp.maximum(z[:, D:] + br_ref[...], 0.0)


def _pre(h, wc, br2, degp):
    return pl.pallas_call(
        _pre_body,
        grid=(NB,),
        in_specs=[
            pl.BlockSpec((RB, D), lambda i: (i, 0)),
            pl.BlockSpec((D, 2 * D), lambda i: (0, 0)),
            pl.BlockSpec((1, D), lambda i: (0, 0)),
            pl.BlockSpec((NC, RB, 16), lambda i: (0, i, 0)),
        ],
        out_specs=[
            pl.BlockSpec((RB, D), lambda i: (i, 0)),
            pl.BlockSpec((RB, D), lambda i: (i, 0)),
        ],
        out_shape=[
            jax.ShapeDtypeStruct((N, D), jnp.float32),
            jax.ShapeDtypeStruct((N, D), jnp.float32),
        ],
    )(h, wc, br2, degp)


def _postA_body(p_ref, y_ref, r_ref, b_ref, degp_ref, z_ref, stats_ref, acc):
    i = pl.program_id(0)
    dis = _dis_block(degp_ref[...])
    agg = (p_ref[0] + p_ref[1] + y_ref[...]) * dis[:, None]
    zb = agg + b_ref[...] + r_ref[...]
    z_ref[...] = zb

    @pl.when(i == 0)
    def _():
        acc[...] = jnp.zeros((8, D), jnp.float32)

    acc[0, :] = acc[0, :] + jnp.sum(zb, axis=0)
    acc[1, :] = acc[1, :] + jnp.sum(zb * zb, axis=0)

    @pl.when(i == NB - 1)
    def _():
        stats_ref[...] = acc[...]


def _postA(p, y, r, b2, degp):
    return pl.pallas_call(
        _postA_body,
        grid=(NB,),
        in_specs=[
            pl.BlockSpec((NC, RB, D), lambda i: (0, i, 0)),
            pl.BlockSpec((RB, D), lambda i: (i, 0)),
            pl.BlockSpec((RB, D), lambda i: (i, 0)),
            pl.BlockSpec((1, D), lambda i: (0, 0)),
            pl.BlockSpec((NC, RB, 16), lambda i: (0, i, 0)),
        ],
        out_specs=[
            pl.BlockSpec((RB, D), lambda i: (i, 0)),
            pl.BlockSpec((8, D), lambda i: (0, 0)),
        ],
        out_shape=[
            jax.ShapeDtypeStruct((N, D), jnp.float32),
            jax.ShapeDtypeStruct((8, D), jnp.float32),
        ],
        scratch_shapes=[pltpu.VMEM((8, D), jnp.float32)],
    )(p, y, r, b2, degp)


def _bn_block(z, stats, g2, be2):
    mean = stats[0, :] * (1.0 / N)
    var = stats[1, :] * (1.0 / N) - mean * mean
    scale = lax.rsqrt(var + EPS) * g2[0, :]
    return (z - mean[None, :]) * scale[None, :] + be2[0, :][None, :]


def _postB_body(z_ref, stats_ref, g_ref, be_ref, h_ref):
    h_ref[...] = _bn_block(z_ref[...], stats_ref[...], g_ref[...], be_ref[...])


def _postB(z, stats, g2, be2):
    return pl.pallas_call(
        _postB_body,
        grid=(NB,),
        in_specs=[
            pl.BlockSpec((RB, D), lambda i: (i, 0)),
            pl.BlockSpec((8, D), lambda i: (0, 0)),
            pl.BlockSpec((1, D), lambda i: (0, 0)),
            pl.BlockSpec((1, D), lambda i: (0, 0)),
        ],
        out_specs=pl.BlockSpec((RB, D), lambda i: (i, 0)),
        out_shape=jax.ShapeDtypeStruct((N, D), jnp.float32),
    )(z, stats, g2, be2)


def _postB2_body(z_ref, stats_ref, g_ref, be_ref, batch_ref, out_ref, acc):
    i = pl.program_id(0)
    hb = _bn_block(z_ref[...], stats_ref[...], g_ref[...], be_ref[...])
    seg = lax.broadcasted_iota(jnp.int32, (RB, G), 1)
    onehot = (batch_ref[...] == seg).astype(jnp.float32)
    pooled = lax.dot_general(onehot, hb, (((0,), (0,)), ((), ())),
                             preferred_element_type=jnp.float32)

    @pl.when(i == 0)
    def _():
        acc[...] = jnp.zeros((G, D), jnp.float32)

    acc[...] = acc[...] + pooled

    @pl.when(i == NB - 1)
    def _():
        out_ref[...] = acc[...]


def _postB2(z, stats, g2, be2, batch2):
    return pl.pallas_call(
        _postB2_body,
        grid=(NB,),
        in_specs=[
            pl.BlockSpec((RB, D), lambda i: (i, 0)),
            pl.BlockSpec((8, D), lambda i: (0, 0)),
            pl.BlockSpec((1, D), lambda i: (0, 0)),
            pl.BlockSpec((1, D), lambda i: (0, 0)),
            pl.BlockSpec((RB, 1), lambda i: (i, 0)),
        ],
        out_specs=pl.BlockSpec((G, D), lambda i: (0, 0)),
        out_shape=jax.ShapeDtypeStruct((G, D), jnp.float32),
        scratch_shapes=[pltpu.VMEM((G, D), jnp.float32)],
    )(z, stats, g2, be2, batch2)


def kernel(x, edge_index, batch,
           W0, b0, Wr0, br0, g0, be0,
           W1, b1, Wr1, br1, g1, be1,
           W2, b2, Wr2, br2, g2, be2):
    src = edge_index[0].astype(jnp.int32)
    dst = edge_index[1].astype(jnp.int32)
    batch2 = batch.astype(jnp.int32).reshape(N, 1)

    degp = _deg(dst)

    layers = [
        (W0, b0, Wr0, br0, g0, be0),
        (W1, b1, Wr1, br1, g1, be1),
        (W2, b2, Wr2, br2, g2, be2),
    ]

    h = x
    out = None
    for li, (W, b, Wr, br, g, be) in enumerate(layers):
        wc = jnp.concatenate([W, Wr], axis=1)
        y, r = _pre(h, wc, br.reshape(1, D), degp)
        p = _agg(y, src, dst)
        z, stats = _postA(p, y, r, b.reshape(1, D), degp)
        if li < 2:
            h = _postB(z, stats, g.reshape(1, D), be.reshape(1, D))
        else:
            out = _postB2(z, stats, g.reshape(1, D), be.reshape(1, D), batch2)
    return out
